# baseline (speedup 1.0000x reference)
import jax
import jax.numpy as jnp
from jax import lax
from jax.experimental import pallas as pl
from jax.experimental.pallas import tpu as pltpu
from jax.experimental.pallas import tpu_sc as plsc

N = 10000
E = 320000
D = 128
ED = 16

NC = 2
NS = 16
NT = NC * NS
EPT = E // NT
NPAD = 10112
RPS = NPAD // NS

_f32 = jnp.float32
_i32 = jnp.int32


def _p0_body(x_ref, w_ref, asd_ref, h_ref, hsd_ref):
    h = jnp.dot(x_ref[...], w_ref[...], preferred_element_type=_f32)
    h_ref[...] = h
    hsd_ref[...] = jnp.dot(h, asd_ref[...], preferred_element_type=_f32)


def _tc_prep(x, W, a_s, a_d):
    asd = jnp.stack([a_s, a_d], axis=1)
    rb = 128
    nb = NPAD // rb
    return pl.pallas_call(
        _p0_body,
        grid=(nb,),
        in_specs=[
            pl.BlockSpec((rb, D), lambda i: (i, 0)),
            pl.BlockSpec((D, D), lambda i: (0, 0)),
            pl.BlockSpec((D, 2), lambda i: (0, 0)),
        ],
        out_specs=[
            pl.BlockSpec((rb, D), lambda i: (i, 0)),
            pl.BlockSpec((rb, 2), lambda i: (i, 0)),
        ],
        out_shape=[
            jax.ShapeDtypeStruct((NPAD, D), _f32),
            jax.ShapeDtypeStruct((NPAD, 2), _f32),
        ],
    )(x, W, asd)


def _p1_body(ea_ref, we1_ref, ae1_ref, we2_ref, ae2_ref, wc1e_ref, bc1_ref,
             eat_ref, c_ref):
    ve1 = jnp.dot(we1_ref[...], ae1_ref[...], preferred_element_type=_f32)
    ve2 = jnp.dot(we2_ref[...], ae2_ref[...], preferred_element_type=_f32)
    ve = jnp.stack([ve1, ve2], axis=0)
    ea = ea_ref[...]
    eat_ref[...] = lax.dot_general(ve, ea, (((1,), (1,)), ((), ())),
                                   preferred_element_type=_f32)
    c_ref[...] = (jnp.dot(ea, wc1e_ref[...], preferred_element_type=_f32)
                  + bc1_ref[...][None, :])


def _tc_edgeprep(edge_attr, We1, ae1, We2, ae2, Wc1e, bc1):
    nb = 100
    eb = E // nb
    return pl.pallas_call(
        _p1_body,
        grid=(nb,),
        in_specs=[
            pl.BlockSpec((eb, ED), lambda i: (i, 0)),
            pl.BlockSpec((ED, D), lambda i: (0, 0)),
            pl.BlockSpec((D,), lambda i: (0,)),
            pl.BlockSpec((ED, D), lambda i: (0, 0)),
            pl.BlockSpec((D,), lambda i: (0,)),
            pl.BlockSpec((ED, D), lambda i: (0, 0)),
            pl.BlockSpec((D,), lambda i: (0,)),
        ],
        out_specs=[
            pl.BlockSpec((2, eb), lambda i: (0, i)),
            pl.BlockSpec((eb, D), lambda i: (i, 0)),
        ],
        out_shape=[
            jax.ShapeDtypeStruct((2, E), _f32),
            jax.ShapeDtypeStruct((E, D), _f32),
        ],
    )(edge_attr, We1, ae1, We2, ae2, Wc1e, bc1)


def _p2_body(num_ref, den_ref, b_ref, w_ref, asd_ref, h_ref, hsd_ref):
    a = num_ref[0] + num_ref[1]
    den = jnp.sum(den_ref[...], axis=0)[:, None]
    x1 = jnp.maximum(a / (den + 1e-16) + b_ref[...][None, :], 0.0)
    h = jnp.dot(x1, w_ref[...], preferred_element_type=_f32)
    h_ref[...] = h
    hsd_ref[...] = jnp.dot(h, asd_ref[...], preferred_element_type=_f32)


def _tc_combine_prep(num, den, b, W, a_s, a_d):
    asd = jnp.stack([a_s, a_d], axis=1)
    rb = 128
    nb = NPAD // rb
    return pl.pallas_call(
        _p2_body,
        grid=(nb,),
        in_specs=[
            pl.BlockSpec((2, rb, D), lambda i: (0, i, 0)),
            pl.BlockSpec((NT, rb), lambda i: (0, i)),
            pl.BlockSpec((D,), lambda i: (0,)),
            pl.BlockSpec((D, D), lambda i: (0, 0)),
            pl.BlockSpec((D, 2), lambda i: (0, 0)),
        ],
        out_specs=[
            pl.BlockSpec((rb, D), lambda i: (i, 0)),
            pl.BlockSpec((rb, 2), lambda i: (i, 0)),
        ],
        out_shape=[
            jax.ShapeDtypeStruct((NPAD, D), _f32),
            jax.ShapeDtypeStruct((NPAD, 2), _f32),
        ],
    )(num, den, b, W, asd)


def _p3_body(num_ref, den_ref, b_ref, wa_ref, wb_ref, a_ref, bb_ref):
    a = num_ref[0] + num_ref[1]
    den = jnp.sum(den_ref[...], axis=0)[:, None]
    x2 = jnp.maximum(a / (den + 1e-16) + b_ref[...][None, :], 0.0)
    a_ref[...] = jnp.dot(x2, wa_ref[...], preferred_element_type=_f32)
    bb_ref[...] = jnp.dot(x2, wb_ref[...], preferred_element_type=_f32)


def _tc_combine_cls(num, den, b, Wc1a, Wc1b):
    rb = 128
    nb = NPAD // rb
    return pl.pallas_call(
        _p3_body,
        grid=(nb,),
        in_specs=[
            pl.BlockSpec((2, rb, D), lambda i: (0, i, 0)),
            pl.BlockSpec((NT, rb), lambda i: (0, i)),
            pl.BlockSpec((D,), lambda i: (0,)),
            pl.BlockSpec((D, D), lambda i: (0, 0)),
            pl.BlockSpec((D, D), lambda i: (0, 0)),
        ],
        out_specs=[
            pl.BlockSpec((rb, D), lambda i: (i, 0)),
            pl.BlockSpec((rb, D), lambda i: (i, 0)),
        ],
        out_shape=[
            jax.ShapeDtypeStruct((NPAD, D), _f32),
            jax.ShapeDtypeStruct((NPAD, D), _f32),
        ],
    )(num, den, b, Wc1a, Wc1b)


CHW = 128
NCHF = EPT // CHW
TAIL = EPT - NCHF * CHW
NDEN = 10016
CHC = 96
NCHC = EPT // CHC
TAILC = EPT - NCHC * CHC

_mesh = plsc.VectorSubcoreMesh(core_axis_name="c", subcore_axis_name="s",
                               num_cores=NC, num_subcores=NS)
_scp = pltpu.CompilerParams(needs_layout_passes=False)


def _w_body(hs_hbm, hd_hbm, ea_hbm, row_hbm, col_hbm, zerosn_hbm,
            w_out, denp_out,
            hs_v, hd_v, den_v,
            ridx0, ridx1, cval0, cval1, eav0, eav1, wv0, wv1,
            isem0, isem1, osem0, osem1):
    cid = lax.axis_index("c")
    sid = lax.axis_index("s")
    wid = cid * NS + sid
    tbase = wid * EPT
    ridx = (ridx0, ridx1)
    cval = (cval0, cval1)
    eav = (eav0, eav1)
    wv = (wv0, wv1)
    isem = (isem0, isem1)
    osem = (osem0, osem1)

    pltpu.sync_copy(hs_hbm, hs_v)
    pltpu.sync_copy(hd_hbm, hd_v)
    pltpu.sync_copy(zerosn_hbm.at[pl.ds(0, NDEN)], den_v)

    def issue_in(j, p):
        eb = tbase + jnp.minimum(j, NCHF - 1) * CHW
        pltpu.async_copy(row_hbm.at[pl.ds(eb, CHW)], ridx[p], isem[p])
        pltpu.async_copy(col_hbm.at[pl.ds(eb, CHW)], cval[p], isem[p])
        pltpu.async_copy(ea_hbm.at[pl.ds(eb, CHW)], eav[p], isem[p])

    def wait_in(p):
        pltpu.make_async_copy(row_hbm.at[pl.ds(0, CHW)], ridx[p],
                              isem[p]).wait()
        pltpu.make_async_copy(col_hbm.at[pl.ds(0, CHW)], cval[p],
                              isem[p]).wait()
        pltpu.make_async_copy(ea_hbm.at[pl.ds(0, CHW)], eav[p],
                              isem[p]).wait()

    def compute(p, ngroups):
        for k in range(ngroups):
            r16 = ridx[p][pl.ds(k * 16, 16)]
            c16 = cval[p][pl.ds(k * 16, 16)]
            hs = plsc.load_gather(hs_v, [r16])
            hd = plsc.load_gather(hd_v, [c16])
            al = hs + hd + eav[p][pl.ds(k * 16, 16)]
            al = jnp.maximum(al, al * 0.2)
            w16 = jnp.exp(al)
            wv[p][pl.ds(k * 16, 16)] = w16
            plsc.addupdate_scatter(den_v, [c16], w16)

    def issue_out(j, p):
        eb = tbase + j * CHW
        pltpu.async_copy(wv[p], w_out.at[pl.ds(eb, CHW)], osem[p])

    def wait_out(p):
        pltpu.make_async_copy(wv[p], w_out.at[pl.ds(0, CHW)], osem[p]).wait()

    issue_in(0, 0)
    issue_in(1, 1)

    def body(t, carry):
        a = 2 * t
        b = a + 1
        wait_in(0)

        @pl.when(t >= 1)
        def _():
            wait_out(0)

        compute(0, CHW // 16)
        issue_out(a, 0)
        issue_in(a + 2, 0)
        wait_in(1)

        @pl.when(t >= 1)
        def _():
            wait_out(1)

        compute(1, CHW // 16)
        issue_out(b, 1)
        issue_in(b + 2, 1)
        return carry

    lax.fori_loop(0, NCHF // 2, body, 0)
    wait_out(0)
    wait_out(1)
    wait_in(0)
    wait_in(1)

    tb = tbase + NCHF * CHW
    pltpu.sync_copy(row_hbm.at[pl.ds(tb, TAIL)], ridx0.at[pl.ds(0, TAIL)])
    pltpu.sync_copy(col_hbm.at[pl.ds(tb, TAIL)], cval0.at[pl.ds(0, TAIL)])
    pltpu.sync_copy(ea_hbm.at[pl.ds(tb, TAIL)], eav0.at[pl.ds(0, TAIL)])
    compute(0, TAIL // 16)
    pltpu.sync_copy(wv0.at[pl.ds(0, TAIL)], w_out.at[pl.ds(tb, TAIL)])

    pltpu.sync_copy(den_v, denp_out.at[pl.ds(wid * NPAD, NDEN)])


def _sc_weights(hs, hd, ea, row, col, zeros_np):
    k = pl.kernel(
        _w_body,
        out_type=(jax.ShapeDtypeStruct((E,), _f32),
                  jax.ShapeDtypeStruct((NT * NPAD,), _f32)),
        mesh=_mesh,
        scratch_types=[
            pltpu.VMEM((N,), _f32),
            pltpu.VMEM((N,), _f32),
            pltpu.VMEM((NDEN,), _f32),
            pltpu.VMEM((CHW,), _i32), pltpu.VMEM((CHW,), _i32),
            pltpu.VMEM((CHW,), _i32), pltpu.VMEM((CHW,), _i32),
            pltpu.VMEM((CHW,), _f32), pltpu.VMEM((CHW,), _f32),
            pltpu.VMEM((CHW,), _f32), pltpu.VMEM((CHW,), _f32),
            pltpu.SemaphoreType.DMA, pltpu.SemaphoreType.DMA,
            pltpu.SemaphoreType.DMA, pltpu.SemaphoreType.DMA,
        ],
        compiler_params=_scp,
    )
    return k(hs, hd, ea, row, col, zeros_np)


def _m_body(h_hbm, w_hbm, row_hbm, col_hbm, zeros_hbm, acc_out,
            ridx0, ridx1, cval0, cval1, scidx0, scidx1, wval0, wval1,
            hrow0, hrow1, tidx_v, acc_sh,
            isem0, isem1, gsem0, gsem1, ssem0, ssem1):
    cid = lax.axis_index("c")
    sid = lax.axis_index("s")
    wid = cid * NS + sid
    tbase = wid * EPT
    ridx = (ridx0, ridx1)
    cval = (cval0, cval1)
    scidx = (scidx0, scidx1)
    wval = (wval0, wval1)
    hrow = (hrow0, hrow1)
    isem = (isem0, isem1)
    gsem = (gsem0, gsem1)
    ssem = (ssem0, ssem1)

    pltpu.sync_copy(zeros_hbm, acc_sh.at[pl.ds(sid * RPS, RPS)])
    plsc.subcore_barrier()

    def issue_in(j, p):
        eb = tbase + jnp.minimum(j, NCHF - 1) * CHW
        pltpu.async_copy(row_hbm.at[pl.ds(eb, CHW)], ridx[p], isem[p])
        pltpu.async_copy(col_hbm.at[pl.ds(eb, CHW)], cval[p], isem[p])
        pltpu.async_copy(w_hbm.at[pl.ds(eb, CHW)], wval[p], isem[p])

    def wait_in(p):
        pltpu.make_async_copy(row_hbm.at[pl.ds(0, CHW)], ridx[p],
                              isem[p]).wait()
        pltpu.make_async_copy(col_hbm.at[pl.ds(0, CHW)], cval[p],
                              isem[p]).wait()
        pltpu.make_async_copy(w_hbm.at[pl.ds(0, CHW)], wval[p],
                              isem[p]).wait()

    def issue_gather(p):
        pltpu.async_copy(h_hbm.at[ridx[p]], hrow[p], gsem[p])

    def wait_gather(p):
        pltpu.make_async_copy(h_hbm.at[ridx[p]], hrow[p], gsem[p]).wait()

    def process(p, j):
        wregs = [wval[p][pl.ds(k * 16, 16)] for k in range(CHW // 16)]
        for k in range(CHW // 16):
            scidx[p][pl.ds(k * 16, 16)] = cval[p][pl.ds(k * 16, 16)]
        issue_in(j + 2, p)
        for k in range(CHW // 16):
            w16 = wregs[k]
            for i in range(16):
                e = k * 16 + i
                sc = w16[i]
                for c in range(D // 16):
                    v = hrow[p][e, pl.ds(c * 16, 16)]
                    hrow[p][e, pl.ds(c * 16, 16)] = v * sc
        pltpu.async_copy(hrow[p], acc_sh.at[scidx[p]], ssem[p], add=True)

    def wait_scatter(p):
        pltpu.make_async_copy(hrow[p], acc_sh.at[scidx[p]], ssem[p]).wait()

    issue_in(0, 0)
    issue_in(1, 1)
    wait_in(0)
    issue_gather(0)

    def body(t, carry):
        a = 2 * t
        b = a + 1
        wait_in(1)

        @pl.when(t >= 1)
        def _():
            wait_scatter(1)

        issue_gather(1)
        wait_gather(0)
        process(0, a)
        wait_in(0)
        wait_scatter(0)
        issue_gather(0)
        wait_gather(1)
        process(1, b)
        return carry

    lax.fori_loop(0, NCHF // 2, body, 0)
    wait_gather(0)
    wait_scatter(1)
    wait_in(1)

    tb = tbase + NCHF * CHW
    pltpu.sync_copy(row_hbm.at[pl.ds(tb, TAIL)], tidx_v)
    pltpu.sync_copy(col_hbm.at[pl.ds(tb, TAIL)], cval0.at[pl.ds(0, TAIL)])
    pltpu.sync_copy(w_hbm.at[pl.ds(tb, TAIL)], wval0.at[pl.ds(0, TAIL)])
    pltpu.async_copy(h_hbm.at[tidx_v], hrow0.at[pl.ds(0, TAIL)],
                     gsem0).wait()
    w16 = wval0[pl.ds(0, 16)]
    for i in range(TAIL):
        sc = w16[i]
        for c in range(D // 16):
            v = hrow0[i, pl.ds(c * 16, 16)]
            hrow0[i, pl.ds(c * 16, 16)] = v * sc
    tidx_v[pl.ds(0, 16)] = cval0[pl.ds(0, 16)]
    pltpu.sync_copy(hrow0.at[pl.ds(0, TAIL)], acc_sh.at[tidx_v], add=True)

    plsc.subcore_barrier()
    pltpu.sync_copy(acc_sh.at[pl.ds(sid * RPS, RPS)],
                    acc_out.at[cid, pl.ds(sid * RPS, RPS)])


def _sc_aggregate(h, w, row, col, zeros_rps):
    k = pl.kernel(
        _m_body,
        out_type=jax.ShapeDtypeStruct((NC, NPAD, D), _f32),
        mesh=_mesh,
        scratch_types=[
            pltpu.VMEM((CHW,), _i32), pltpu.VMEM((CHW,), _i32),
            pltpu.VMEM((CHW,), _i32), pltpu.VMEM((CHW,), _i32),
            pltpu.VMEM((CHW,), _i32), pltpu.VMEM((CHW,), _i32),
            pltpu.VMEM((CHW,), _f32), pltpu.VMEM((CHW,), _f32),
            pltpu.VMEM((CHW, D), _f32), pltpu.VMEM((CHW, D), _f32),
            pltpu.VMEM((TAIL,), _i32),
            pltpu.VMEM_SHARED((NPAD, D), _f32),
            pltpu.SemaphoreType.DMA, pltpu.SemaphoreType.DMA,
            pltpu.SemaphoreType.DMA, pltpu.SemaphoreType.DMA,
            pltpu.SemaphoreType.DMA, pltpu.SemaphoreType.DMA,
        ],
        compiler_params=_scp,
    )
    return k(h, w, row, col, zeros_rps)


def _cls_body(a_hbm, b_hbm, c_hbm, wc2_hbm, row_hbm, col_hbm, out_hbm,
              wc2_v, tbuf_v, ridx0, ridx1, cval0, cval1,
              arow0, arow1, brow0, brow1, crow0, crow1, outv0, outv1,
              isem0, isem1, gasem0, gasem1, gbsem0, gbsem1,
              csem0, csem1, osem0, osem1):
    cid = lax.axis_index("c")
    sid = lax.axis_index("s")
    wid = cid * NS + sid
    tbase = wid * EPT
    ridx = (ridx0, ridx1)
    cval = (cval0, cval1)
    arow = (arow0, arow1)
    brow = (brow0, brow1)
    crow = (crow0, crow1)
    outv = (outv0, outv1)
    isem = (isem0, isem1)
    gasem = (gasem0, gasem1)
    gbsem = (gbsem0, gbsem1)
    csem = (csem0, csem1)
    osem = (osem0, osem1)

    pltpu.sync_copy(wc2_hbm, wc2_v)
    lane = lax.iota(_i32, 16)
    wc2 = [wc2_v[pl.ds(c * 16, 16)] for c in range(D // 16)]
    jsplat = [jnp.full((16,), j, _i32) for j in range(16)]

    def issue_in(j, p):
        eb = tbase + jnp.minimum(j, NCHC - 1) * CHC
        pltpu.async_copy(row_hbm.at[pl.ds(eb, CHC)], ridx[p], isem[p])
        pltpu.async_copy(col_hbm.at[pl.ds(eb, CHC)], cval[p], isem[p])

    def wait_in(p):
        pltpu.make_async_copy(row_hbm.at[pl.ds(0, CHC)], ridx[p],
                              isem[p]).wait()
        pltpu.make_async_copy(col_hbm.at[pl.ds(0, CHC)], cval[p],
                              isem[p]).wait()

    def issue_gathers(j, p):
        eb = tbase + jnp.minimum(j, NCHC - 1) * CHC
        pltpu.async_copy(a_hbm.at[ridx[p]], arow[p], gasem[p])
        pltpu.async_copy(b_hbm.at[cval[p]], brow[p], gbsem[p])
        pltpu.async_copy(c_hbm.at[pl.ds(eb, CHC)], crow[p], csem[p])

    def wait_gathers(p):
        pltpu.make_async_copy(a_hbm.at[ridx[p]], arow[p], gasem[p]).wait()
        pltpu.make_async_copy(b_hbm.at[cval[p]], brow[p], gbsem[p]).wait()
        pltpu.make_async_copy(c_hbm.at[pl.ds(0, CHC)], crow[p],
                              csem[p]).wait()

    def compute(p, ngroups):
        for k in range(ngroups):
            for i in range(16):
                e = k * 16 + i
                acc = jnp.zeros((16,), _f32)
                for c in range(D // 16):
                    g = (arow[p][e, pl.ds(c * 16, 16)]
                         + brow[p][e, pl.ds(c * 16, 16)]
                         + crow[p][e, pl.ds(c * 16, 16)])
                    acc = acc + jnp.maximum(g, 0.0) * wc2[c]
                tbuf_v[i, pl.ds(0, 16)] = acc
            res = plsc.load_gather(tbuf_v, [lane, jsplat[0]])
            for j in range(1, 16):
                res = res + plsc.load_gather(tbuf_v, [lane, jsplat[j]])
            outv[p][pl.ds(k * 16, 16)] = res

    def issue_out(j, p):
        eb = tbase + j * CHC
        pltpu.async_copy(outv[p], out_hbm.at[pl.ds(eb, CHC)], osem[p])

    def wait_out(p):
        pltpu.make_async_copy(outv[p], out_hbm.at[pl.ds(0, CHC)],
                              osem[p]).wait()

    issue_in(0, 0)
    issue_in(1, 1)
    wait_in(0)
    issue_gathers(0, 0)

    def body(t, carry):
        a = 2 * t
        b = a + 1
        wait_in(1)
        issue_gathers(b, 1)
        wait_gathers(0)
        issue_in(a + 2, 0)

        @pl.when(t >= 1)
        def _():
            wait_out(0)

        compute(0, CHC // 16)
        issue_out(a, 0)
        wait_in(0)
        issue_gathers(a + 2, 0)
        wait_gathers(1)
        issue_in(b + 2, 1)

        @pl.when(t >= 1)
        def _():
            wait_out(1)

        compute(1, CHC // 16)
        issue_out(b, 1)
        return carry

    lax.fori_loop(0, NCHC // 2, body, 0)
    wait_gathers(0)
    wait_out(0)
    wait_out(1)
    wait_in(1)

    tb = tbase + NCHC * CHC
    pltpu.sync_copy(row_hbm.at[pl.ds(tb, TAILC)], ridx0.at[pl.ds(0, TAILC)])
    pltpu.sync_copy(col_hbm.at[pl.ds(tb, TAILC)], cval0.at[pl.ds(0, TAILC)])
    cpa = pltpu.async_copy(a_hbm.at[ridx0.at[pl.ds(0, TAILC)]],
                           arow0.at[pl.ds(0, TAILC)], gasem0)
    cpb = pltpu.async_copy(b_hbm.at[cval0.at[pl.ds(0, TAILC)]],
                           brow0.at[pl.ds(0, TAILC)], gbsem0)
    pltpu.sync_copy(c_hbm.at[pl.ds(tb, TAILC)], crow0.at[pl.ds(0, TAILC)])
    cpa.wait()
    cpb.wait()
    compute(0, TAILC // 16)
    pltpu.sync_copy(outv0.at[pl.ds(0, TAILC)], out_hbm.at[pl.ds(tb, TAILC)])


def _sc_classifier(A, B, C, wc2, row, col):
    k = pl.kernel(
        _cls_body,
        out_type=jax.ShapeDtypeStruct((E,), _f32),
        mesh=_mesh,
        scratch_types=[
            pltpu.VMEM((D,), _f32),
            pltpu.VMEM((16, 16), _f32),
            pltpu.VMEM((CHC,), _i32), pltpu.VMEM((CHC,), _i32),
            pltpu.VMEM((CHC,), _i32), pltpu.VMEM((CHC,), _i32),
            pltpu.VMEM((CHC, D), _f32), pltpu.VMEM((CHC, D), _f32),
            pltpu.VMEM((CHC, D), _f32), pltpu.VMEM((CHC, D), _f32),
            pltpu.VMEM((CHC, D), _f32), pltpu.VMEM((CHC, D), _f32),
            pltpu.VMEM((CHC,), _f32), pltpu.VMEM((CHC,), _f32),
            pltpu.SemaphoreType.DMA, pltpu.SemaphoreType.DMA,
            pltpu.SemaphoreType.DMA, pltpu.SemaphoreType.DMA,
            pltpu.SemaphoreType.DMA, pltpu.SemaphoreType.DMA,
            pltpu.SemaphoreType.DMA, pltpu.SemaphoreType.DMA,
            pltpu.SemaphoreType.DMA, pltpu.SemaphoreType.DMA,
        ],
        compiler_params=_scp,
    )
    return k(A, B, C, wc2, row, col)


def kernel(x, edge_index, edge_attr, W1, as1, ad1, We1, ae1, b1,
           W2, as2, ad2, We2, ae2, b2, Wc1, bc1, Wc2, bc2):
    row = edge_index[0].astype(_i32)
    col = edge_index[1].astype(_i32)
    zeros_rps = jnp.zeros((RPS, D), _f32)
    zeros_np = jnp.zeros((NPAD,), _f32)
    x = jnp.pad(x, ((0, NPAD - N), (0, 0)))

    eaT, C = _tc_edgeprep(edge_attr, We1, ae1, We2, ae2, Wc1[2 * D:], bc1)

    h1, hsd1 = _tc_prep(x, W1, as1, ad1)
    w1, denp1 = _sc_weights(hsd1[:N, 0], hsd1[:N, 1], eaT[0], row, col,
                            zeros_np)
    acc1 = _sc_aggregate(h1, w1, row, col, zeros_rps)
    den1 = denp1.reshape(NT, NPAD)

    h2, hsd2 = _tc_combine_prep(acc1, den1, b1, W2, as2, ad2)
    w2, denp2 = _sc_weights(hsd2[:N, 0], hsd2[:N, 1], eaT[1], row, col,
                            zeros_np)
    acc2 = _sc_aggregate(h2, w2, row, col, zeros_rps)
    den2 = denp2.reshape(NT, NPAD)

    A, B = _tc_combine_cls(acc2, den2, b2, Wc1[:D], Wc1[D:2 * D])
    out = _sc_classifier(A, B, C, Wc2[:, 0], row, col)
    return out + bc2[0]


# speedup vs baseline: 1.0211x; 1.0211x over previous
import jax
import jax.numpy as jnp
from jax import lax
from jax.experimental import pallas as pl
from jax.experimental.pallas import tpu as pltpu
from jax.experimental.pallas import tpu_sc as plsc

N = 10000
E = 320000
D = 128
ED = 16

NC = 2
NS = 16
NT = NC * NS
EPT = E // NT
NPAD = 10112
RPS = NPAD // NS

_f32 = jnp.float32
_i32 = jnp.int32


def _p0_body(x_ref, w_ref, asd_ref, h_ref, hsd_ref):
    h = jnp.dot(x_ref[...], w_ref[...], preferred_element_type=_f32)
    h_ref[...] = h
    hsd_ref[...] = jnp.dot(h, asd_ref[...], preferred_element_type=_f32)


def _tc_prep(x, W, a_s, a_d):
    asd = jnp.stack([a_s, a_d], axis=1)
    rb = 128
    nb = NPAD // rb
    return pl.pallas_call(
        _p0_body,
        grid=(nb,),
        in_specs=[
            pl.BlockSpec((rb, D), lambda i: (i, 0)),
            pl.BlockSpec((D, D), lambda i: (0, 0)),
            pl.BlockSpec((D, 2), lambda i: (0, 0)),
        ],
        out_specs=[
            pl.BlockSpec((rb, D), lambda i: (i, 0)),
            pl.BlockSpec((rb, 2), lambda i: (i, 0)),
        ],
        out_shape=[
            jax.ShapeDtypeStruct((NPAD, D), _f32),
            jax.ShapeDtypeStruct((NPAD, 2), _f32),
        ],
    )(x, W, asd)


def _p1_body(ea_ref, we1_ref, ae1_ref, we2_ref, ae2_ref, wc1e_ref, bc1_ref,
             eat_ref, c_ref):
    ve1 = jnp.dot(we1_ref[...], ae1_ref[...], preferred_element_type=_f32)
    ve2 = jnp.dot(we2_ref[...], ae2_ref[...], preferred_element_type=_f32)
    ve = jnp.stack([ve1, ve2], axis=0)
    ea = ea_ref[...]
    eat_ref[...] = lax.dot_general(ve, ea, (((1,), (1,)), ((), ())),
                                   preferred_element_type=_f32)
    c_ref[...] = (jnp.dot(ea, wc1e_ref[...], preferred_element_type=_f32)
                  + bc1_ref[...][None, :])


def _tc_edgeprep(edge_attr, We1, ae1, We2, ae2, Wc1e, bc1):
    nb = 100
    eb = E // nb
    return pl.pallas_call(
        _p1_body,
        grid=(nb,),
        in_specs=[
            pl.BlockSpec((eb, ED), lambda i: (i, 0)),
            pl.BlockSpec((ED, D), lambda i: (0, 0)),
            pl.BlockSpec((D,), lambda i: (0,)),
            pl.BlockSpec((ED, D), lambda i: (0, 0)),
            pl.BlockSpec((D,), lambda i: (0,)),
            pl.BlockSpec((ED, D), lambda i: (0, 0)),
            pl.BlockSpec((D,), lambda i: (0,)),
        ],
        out_specs=[
            pl.BlockSpec((2, eb), lambda i: (0, i)),
            pl.BlockSpec((eb, D), lambda i: (i, 0)),
        ],
        out_shape=[
            jax.ShapeDtypeStruct((2, E), _f32),
            jax.ShapeDtypeStruct((E, D), _f32),
        ],
    )(edge_attr, We1, ae1, We2, ae2, Wc1e, bc1)


def _p2_body(num_ref, den_ref, b_ref, w_ref, asd_ref, h_ref, hsd_ref):
    a = num_ref[0] + num_ref[1]
    den = jnp.sum(den_ref[...], axis=0)[:, None]
    x1 = jnp.maximum(a / (den + 1e-16) + b_ref[...][None, :], 0.0)
    h = jnp.dot(x1, w_ref[...], preferred_element_type=_f32)
    h_ref[...] = h
    hsd_ref[...] = jnp.dot(h, asd_ref[...], preferred_element_type=_f32)


def _tc_combine_prep(num, den, b, W, a_s, a_d):
    asd = jnp.stack([a_s, a_d], axis=1)
    rb = 128
    nb = NPAD // rb
    return pl.pallas_call(
        _p2_body,
        grid=(nb,),
        in_specs=[
            pl.BlockSpec((2, rb, D), lambda i: (0, i, 0)),
            pl.BlockSpec((NT, rb), lambda i: (0, i)),
            pl.BlockSpec((D,), lambda i: (0,)),
            pl.BlockSpec((D, D), lambda i: (0, 0)),
            pl.BlockSpec((D, 2), lambda i: (0, 0)),
        ],
        out_specs=[
            pl.BlockSpec((rb, D), lambda i: (i, 0)),
            pl.BlockSpec((rb, 2), lambda i: (i, 0)),
        ],
        out_shape=[
            jax.ShapeDtypeStruct((NPAD, D), _f32),
            jax.ShapeDtypeStruct((NPAD, 2), _f32),
        ],
    )(num, den, b, W, asd)


def _p3_body(num_ref, den_ref, b_ref, wa_ref, wb_ref, a_ref, bb_ref):
    a = num_ref[0] + num_ref[1]
    den = jnp.sum(den_ref[...], axis=0)[:, None]
    x2 = jnp.maximum(a / (den + 1e-16) + b_ref[...][None, :], 0.0)
    a_ref[...] = jnp.dot(x2, wa_ref[...], preferred_element_type=_f32)
    bb_ref[...] = jnp.dot(x2, wb_ref[...], preferred_element_type=_f32)


def _tc_combine_cls(num, den, b, Wc1a, Wc1b):
    rb = 128
    nb = NPAD // rb
    return pl.pallas_call(
        _p3_body,
        grid=(nb,),
        in_specs=[
            pl.BlockSpec((2, rb, D), lambda i: (0, i, 0)),
            pl.BlockSpec((NT, rb), lambda i: (0, i)),
            pl.BlockSpec((D,), lambda i: (0,)),
            pl.BlockSpec((D, D), lambda i: (0, 0)),
            pl.BlockSpec((D, D), lambda i: (0, 0)),
        ],
        out_specs=[
            pl.BlockSpec((rb, D), lambda i: (i, 0)),
            pl.BlockSpec((rb, D), lambda i: (i, 0)),
        ],
        out_shape=[
            jax.ShapeDtypeStruct((NPAD, D), _f32),
            jax.ShapeDtypeStruct((NPAD, D), _f32),
        ],
    )(num, den, b, Wc1a, Wc1b)


CHW = 128
NCHF = EPT // CHW
TAIL = EPT - NCHF * CHW
NDEN = 10016
CHC = 64
NCHC = EPT // CHC
TAILC = EPT - NCHC * CHC

_mesh = plsc.VectorSubcoreMesh(core_axis_name="c", subcore_axis_name="s",
                               num_cores=NC, num_subcores=NS)
_scp = pltpu.CompilerParams(needs_layout_passes=False)


def _w_body(hs_hbm, hd_hbm, ea_hbm, row_hbm, col_hbm, zerosn_hbm,
            w_out, denp_out,
            hs_v, hd_v, den_v,
            ridx0, ridx1, cval0, cval1, eav0, eav1, wv0, wv1,
            isem0, isem1, osem0, osem1):
    cid = lax.axis_index("c")
    sid = lax.axis_index("s")
    wid = cid * NS + sid
    tbase = wid * EPT
    ridx = (ridx0, ridx1)
    cval = (cval0, cval1)
    eav = (eav0, eav1)
    wv = (wv0, wv1)
    isem = (isem0, isem1)
    osem = (osem0, osem1)

    pltpu.sync_copy(hs_hbm, hs_v)
    pltpu.sync_copy(hd_hbm, hd_v)
    pltpu.sync_copy(zerosn_hbm.at[pl.ds(0, NDEN)], den_v)

    def issue_in(j, p):
        eb = tbase + jnp.minimum(j, NCHF - 1) * CHW
        pltpu.async_copy(row_hbm.at[pl.ds(eb, CHW)], ridx[p], isem[p])
        pltpu.async_copy(col_hbm.at[pl.ds(eb, CHW)], cval[p], isem[p])
        pltpu.async_copy(ea_hbm.at[pl.ds(eb, CHW)], eav[p], isem[p])

    def wait_in(p):
        pltpu.make_async_copy(row_hbm.at[pl.ds(0, CHW)], ridx[p],
                              isem[p]).wait()
        pltpu.make_async_copy(col_hbm.at[pl.ds(0, CHW)], cval[p],
                              isem[p]).wait()
        pltpu.make_async_copy(ea_hbm.at[pl.ds(0, CHW)], eav[p],
                              isem[p]).wait()

    def compute(p, ngroups):
        for k in range(ngroups):
            r16 = ridx[p][pl.ds(k * 16, 16)]
            c16 = cval[p][pl.ds(k * 16, 16)]
            hs = plsc.load_gather(hs_v, [r16])
            hd = plsc.load_gather(hd_v, [c16])
            al = hs + hd + eav[p][pl.ds(k * 16, 16)]
            al = jnp.maximum(al, al * 0.2)
            w16 = jnp.exp(al)
            wv[p][pl.ds(k * 16, 16)] = w16
            plsc.addupdate_scatter(den_v, [c16], w16)

    def issue_out(j, p):
        eb = tbase + j * CHW
        pltpu.async_copy(wv[p], w_out.at[pl.ds(eb, CHW)], osem[p])

    def wait_out(p):
        pltpu.make_async_copy(wv[p], w_out.at[pl.ds(0, CHW)], osem[p]).wait()

    issue_in(0, 0)
    issue_in(1, 1)

    def body(t, carry):
        a = 2 * t
        b = a + 1
        wait_in(0)

        @pl.when(t >= 1)
        def _():
            wait_out(0)

        compute(0, CHW // 16)
        issue_out(a, 0)
        issue_in(a + 2, 0)
        wait_in(1)

        @pl.when(t >= 1)
        def _():
            wait_out(1)

        compute(1, CHW // 16)
        issue_out(b, 1)
        issue_in(b + 2, 1)
        return carry

    lax.fori_loop(0, NCHF // 2, body, 0)
    wait_out(0)
    wait_out(1)
    wait_in(0)
    wait_in(1)

    tb = tbase + NCHF * CHW
    pltpu.sync_copy(row_hbm.at[pl.ds(tb, TAIL)], ridx0.at[pl.ds(0, TAIL)])
    pltpu.sync_copy(col_hbm.at[pl.ds(tb, TAIL)], cval0.at[pl.ds(0, TAIL)])
    pltpu.sync_copy(ea_hbm.at[pl.ds(tb, TAIL)], eav0.at[pl.ds(0, TAIL)])
    compute(0, TAIL // 16)
    pltpu.sync_copy(wv0.at[pl.ds(0, TAIL)], w_out.at[pl.ds(tb, TAIL)])

    pltpu.sync_copy(den_v, denp_out.at[pl.ds(wid * NPAD, NDEN)])


def _sc_weights(hs, hd, ea, row, col, zeros_np):
    k = pl.kernel(
        _w_body,
        out_type=(jax.ShapeDtypeStruct((E,), _f32),
                  jax.ShapeDtypeStruct((NT * NPAD,), _f32)),
        mesh=_mesh,
        scratch_types=[
            pltpu.VMEM((N,), _f32),
            pltpu.VMEM((N,), _f32),
            pltpu.VMEM((NDEN,), _f32),
            pltpu.VMEM((CHW,), _i32), pltpu.VMEM((CHW,), _i32),
            pltpu.VMEM((CHW,), _i32), pltpu.VMEM((CHW,), _i32),
            pltpu.VMEM((CHW,), _f32), pltpu.VMEM((CHW,), _f32),
            pltpu.VMEM((CHW,), _f32), pltpu.VMEM((CHW,), _f32),
            pltpu.SemaphoreType.DMA, pltpu.SemaphoreType.DMA,
            pltpu.SemaphoreType.DMA, pltpu.SemaphoreType.DMA,
        ],
        compiler_params=_scp,
    )
    return k(hs, hd, ea, row, col, zeros_np)


def _m_body(h_hbm, w_hbm, row_hbm, col_hbm, zeros_hbm, acc_out,
            ridx0, ridx1, cval0, cval1, scidx0, scidx1, wval0, wval1,
            hrow0, hrow1, tidx_v, acc_sh,
            isem0, isem1, gsem0, gsem1, ssem0, ssem1):
    cid = lax.axis_index("c")
    sid = lax.axis_index("s")
    wid = cid * NS + sid
    tbase = wid * EPT
    ridx = (ridx0, ridx1)
    cval = (cval0, cval1)
    scidx = (scidx0, scidx1)
    wval = (wval0, wval1)
    hrow = (hrow0, hrow1)
    isem = (isem0, isem1)
    gsem = (gsem0, gsem1)
    ssem = (ssem0, ssem1)

    pltpu.sync_copy(zeros_hbm, acc_sh.at[pl.ds(sid * RPS, RPS)])
    plsc.subcore_barrier()

    def issue_in(j, p):
        eb = tbase + jnp.minimum(j, NCHF - 1) * CHW
        pltpu.async_copy(row_hbm.at[pl.ds(eb, CHW)], ridx[p], isem[p])
        pltpu.async_copy(col_hbm.at[pl.ds(eb, CHW)], cval[p], isem[p])
        pltpu.async_copy(w_hbm.at[pl.ds(eb, CHW)], wval[p], isem[p])

    def wait_in(p):
        pltpu.make_async_copy(row_hbm.at[pl.ds(0, CHW)], ridx[p],
                              isem[p]).wait()
        pltpu.make_async_copy(col_hbm.at[pl.ds(0, CHW)], cval[p],
                              isem[p]).wait()
        pltpu.make_async_copy(w_hbm.at[pl.ds(0, CHW)], wval[p],
                              isem[p]).wait()

    def issue_gather(p):
        pltpu.async_copy(h_hbm.at[ridx[p]], hrow[p], gsem[p])

    def wait_gather(p):
        pltpu.make_async_copy(h_hbm.at[ridx[p]], hrow[p], gsem[p]).wait()

    def process(p, j):
        wregs = [wval[p][pl.ds(k * 16, 16)] for k in range(CHW // 16)]
        for k in range(CHW // 16):
            scidx[p][pl.ds(k * 16, 16)] = cval[p][pl.ds(k * 16, 16)]
        issue_in(j + 2, p)
        for k in range(CHW // 16):
            w16 = wregs[k]
            for i in range(16):
                e = k * 16 + i
                sc = w16[i]
                for c in range(D // 16):
                    v = hrow[p][e, pl.ds(c * 16, 16)]
                    hrow[p][e, pl.ds(c * 16, 16)] = v * sc
        pltpu.async_copy(hrow[p], acc_sh.at[scidx[p]], ssem[p], add=True)

    def wait_scatter(p):
        pltpu.make_async_copy(hrow[p], acc_sh.at[scidx[p]], ssem[p]).wait()

    issue_in(0, 0)
    issue_in(1, 1)
    wait_in(0)
    issue_gather(0)

    def body(t, carry):
        a = 2 * t
        b = a + 1
        wait_in(1)

        @pl.when(t >= 1)
        def _():
            wait_scatter(1)

        issue_gather(1)
        wait_gather(0)
        process(0, a)
        wait_in(0)
        wait_scatter(0)
        issue_gather(0)
        wait_gather(1)
        process(1, b)
        return carry

    lax.fori_loop(0, NCHF // 2, body, 0)
    wait_gather(0)
    wait_scatter(1)
    wait_in(1)

    tb = tbase + NCHF * CHW
    pltpu.sync_copy(row_hbm.at[pl.ds(tb, TAIL)], tidx_v)
    pltpu.sync_copy(col_hbm.at[pl.ds(tb, TAIL)], cval0.at[pl.ds(0, TAIL)])
    pltpu.sync_copy(w_hbm.at[pl.ds(tb, TAIL)], wval0.at[pl.ds(0, TAIL)])
    pltpu.async_copy(h_hbm.at[tidx_v], hrow0.at[pl.ds(0, TAIL)],
                     gsem0).wait()
    w16 = wval0[pl.ds(0, 16)]
    for i in range(TAIL):
        sc = w16[i]
        for c in range(D // 16):
            v = hrow0[i, pl.ds(c * 16, 16)]
            hrow0[i, pl.ds(c * 16, 16)] = v * sc
    tidx_v[pl.ds(0, 16)] = cval0[pl.ds(0, 16)]
    pltpu.sync_copy(hrow0.at[pl.ds(0, TAIL)], acc_sh.at[tidx_v], add=True)

    plsc.subcore_barrier()
    pltpu.sync_copy(acc_sh.at[pl.ds(sid * RPS, RPS)],
                    acc_out.at[cid, pl.ds(sid * RPS, RPS)])


def _sc_aggregate(h, w, row, col, zeros_rps):
    k = pl.kernel(
        _m_body,
        out_type=jax.ShapeDtypeStruct((NC, NPAD, D), _f32),
        mesh=_mesh,
        scratch_types=[
            pltpu.VMEM((CHW,), _i32), pltpu.VMEM((CHW,), _i32),
            pltpu.VMEM((CHW,), _i32), pltpu.VMEM((CHW,), _i32),
            pltpu.VMEM((CHW,), _i32), pltpu.VMEM((CHW,), _i32),
            pltpu.VMEM((CHW,), _f32), pltpu.VMEM((CHW,), _f32),
            pltpu.VMEM((CHW, D), _f32), pltpu.VMEM((CHW, D), _f32),
            pltpu.VMEM((TAIL,), _i32),
            pltpu.VMEM_SHARED((NPAD, D), _f32),
            pltpu.SemaphoreType.DMA, pltpu.SemaphoreType.DMA,
            pltpu.SemaphoreType.DMA, pltpu.SemaphoreType.DMA,
            pltpu.SemaphoreType.DMA, pltpu.SemaphoreType.DMA,
        ],
        compiler_params=_scp,
    )
    return k(h, w, row, col, zeros_rps)


def _cls_body(a_hbm, b_hbm, c_hbm, wc2_hbm, row_hbm, col_hbm, out_hbm,
              wc2_v, tbuf_v, ridx0, ridx1, cval0, cval1,
              arow0, arow1, brow0, brow1, crow0, crow1, outv0, outv1,
              isem0, isem1, gasem0, gasem1, gbsem0, gbsem1,
              csem0, csem1, osem0, osem1):
    cid = lax.axis_index("c")
    sid = lax.axis_index("s")
    wid = cid * NS + sid
    tbase = wid * EPT
    ridx = (ridx0, ridx1)
    cval = (cval0, cval1)
    arow = (arow0, arow1)
    brow = (brow0, brow1)
    crow = (crow0, crow1)
    outv = (outv0, outv1)
    isem = (isem0, isem1)
    gasem = (gasem0, gasem1)
    gbsem = (gbsem0, gbsem1)
    csem = (csem0, csem1)
    osem = (osem0, osem1)

    pltpu.sync_copy(wc2_hbm, wc2_v)
    lane = lax.iota(_i32, 16)
    wc2 = [wc2_v[pl.ds(c * 16, 16)] for c in range(D // 16)]
    jsplat = [jnp.full((16,), j, _i32) for j in range(16)]

    def issue_in(j, p):
        eb = tbase + jnp.minimum(j, NCHC - 1) * CHC
        pltpu.async_copy(row_hbm.at[pl.ds(eb, CHC)], ridx[p], isem[p])
        pltpu.async_copy(col_hbm.at[pl.ds(eb, CHC)], cval[p], isem[p])

    def wait_in(p):
        pltpu.make_async_copy(row_hbm.at[pl.ds(0, CHC)], ridx[p],
                              isem[p]).wait()
        pltpu.make_async_copy(col_hbm.at[pl.ds(0, CHC)], cval[p],
                              isem[p]).wait()

    def issue_gathers(j, p):
        eb = tbase + jnp.minimum(j, NCHC - 1) * CHC
        pltpu.async_copy(a_hbm.at[ridx[p]], arow[p], gasem[p])
        pltpu.async_copy(b_hbm.at[cval[p]], brow[p], gbsem[p])
        pltpu.async_copy(c_hbm.at[pl.ds(eb, CHC)], crow[p], csem[p])

    def wait_gathers(p):
        pltpu.make_async_copy(a_hbm.at[ridx[p]], arow[p], gasem[p]).wait()
        pltpu.make_async_copy(b_hbm.at[cval[p]], brow[p], gbsem[p]).wait()
        pltpu.make_async_copy(c_hbm.at[pl.ds(0, CHC)], crow[p],
                              csem[p]).wait()

    def compute(p, ngroups):
        for k in range(ngroups):
            for i in range(16):
                e = k * 16 + i
                acc = jnp.zeros((16,), _f32)
                for c in range(D // 16):
                    g = (arow[p][e, pl.ds(c * 16, 16)]
                         + brow[p][e, pl.ds(c * 16, 16)]
                         + crow[p][e, pl.ds(c * 16, 16)])
                    acc = acc + jnp.maximum(g, 0.0) * wc2[c]
                tbuf_v[i, pl.ds(0, 16)] = acc
            res = plsc.load_gather(tbuf_v, [lane, jsplat[0]])
            for j in range(1, 16):
                res = res + plsc.load_gather(tbuf_v, [lane, jsplat[j]])
            outv[p][pl.ds(k * 16, 16)] = res

    def issue_out(j, p):
        eb = tbase + j * CHC
        pltpu.async_copy(outv[p], out_hbm.at[pl.ds(eb, CHC)], osem[p])

    def wait_out(p):
        pltpu.make_async_copy(outv[p], out_hbm.at[pl.ds(0, CHC)],
                              osem[p]).wait()

    issue_in(0, 0)
    issue_in(1, 1)
    wait_in(0)
    issue_gathers(0, 0)

    def body(t, carry):
        a = 2 * t
        b = a + 1
        wait_in(1)
        issue_gathers(b, 1)
        wait_gathers(0)
        issue_in(a + 2, 0)

        @pl.when(t >= 1)
        def _():
            wait_out(0)

        compute(0, CHC // 16)
        issue_out(a, 0)
        wait_in(0)
        issue_gathers(a + 2, 0)
        wait_gathers(1)
        issue_in(b + 2, 1)

        @pl.when(t >= 1)
        def _():
            wait_out(1)

        compute(1, CHC // 16)
        issue_out(b, 1)
        return carry

    lax.fori_loop(0, NCHC // 2, body, 0)
    wait_gathers(0)
    wait_out(0)
    wait_out(1)
    wait_in(1)

    tb = tbase + NCHC * CHC
    pltpu.sync_copy(row_hbm.at[pl.ds(tb, TAILC)], ridx0.at[pl.ds(0, TAILC)])
    pltpu.sync_copy(col_hbm.at[pl.ds(tb, TAILC)], cval0.at[pl.ds(0, TAILC)])
    cpa = pltpu.async_copy(a_hbm.at[ridx0.at[pl.ds(0, TAILC)]],
                           arow0.at[pl.ds(0, TAILC)], gasem0)
    cpb = pltpu.async_copy(b_hbm.at[cval0.at[pl.ds(0, TAILC)]],
                           brow0.at[pl.ds(0, TAILC)], gbsem0)
    pltpu.sync_copy(c_hbm.at[pl.ds(tb, TAILC)], crow0.at[pl.ds(0, TAILC)])
    cpa.wait()
    cpb.wait()
    compute(0, TAILC // 16)
    pltpu.sync_copy(outv0.at[pl.ds(0, TAILC)], out_hbm.at[pl.ds(tb, TAILC)])


def _sc_classifier(A, B, C, wc2, row, col):
    k = pl.kernel(
        _cls_body,
        out_type=jax.ShapeDtypeStruct((E,), _f32),
        mesh=_mesh,
        scratch_types=[
            pltpu.VMEM((D,), _f32),
            pltpu.VMEM((16, 16), _f32),
            pltpu.VMEM((CHC,), _i32), pltpu.VMEM((CHC,), _i32),
            pltpu.VMEM((CHC,), _i32), pltpu.VMEM((CHC,), _i32),
            pltpu.VMEM((CHC, D), _f32), pltpu.VMEM((CHC, D), _f32),
            pltpu.VMEM((CHC, D), _f32), pltpu.VMEM((CHC, D), _f32),
            pltpu.VMEM((CHC, D), _f32), pltpu.VMEM((CHC, D), _f32),
            pltpu.VMEM((CHC,), _f32), pltpu.VMEM((CHC,), _f32),
            pltpu.SemaphoreType.DMA, pltpu.SemaphoreType.DMA,
            pltpu.SemaphoreType.DMA, pltpu.SemaphoreType.DMA,
            pltpu.SemaphoreType.DMA, pltpu.SemaphoreType.DMA,
            pltpu.SemaphoreType.DMA, pltpu.SemaphoreType.DMA,
            pltpu.SemaphoreType.DMA, pltpu.SemaphoreType.DMA,
        ],
        compiler_params=_scp,
    )
    return k(A, B, C, wc2, row, col)


def kernel(x, edge_index, edge_attr, W1, as1, ad1, We1, ae1, b1,
           W2, as2, ad2, We2, ae2, b2, Wc1, bc1, Wc2, bc2):
    row = edge_index[0].astype(_i32)
    col = edge_index[1].astype(_i32)
    zeros_rps = jnp.zeros((RPS, D), _f32)
    zeros_np = jnp.zeros((NPAD,), _f32)
    x = jnp.pad(x, ((0, NPAD - N), (0, 0)))

    eaT, C = _tc_edgeprep(edge_attr, We1, ae1, We2, ae2, Wc1[2 * D:], bc1)

    h1, hsd1 = _tc_prep(x, W1, as1, ad1)
    w1, denp1 = _sc_weights(hsd1[:N, 0], hsd1[:N, 1], eaT[0], row, col,
                            zeros_np)
    acc1 = _sc_aggregate(h1, w1, row, col, zeros_rps)
    den1 = denp1.reshape(NT, NPAD)

    h2, hsd2 = _tc_combine_prep(acc1, den1, b1, W2, as2, ad2)
    w2, denp2 = _sc_weights(hsd2[:N, 0], hsd2[:N, 1], eaT[1], row, col,
                            zeros_np)
    acc2 = _sc_aggregate(h2, w2, row, col, zeros_rps)
    den2 = denp2.reshape(NT, NPAD)

    A, B = _tc_combine_cls(acc2, den2, b2, Wc1[:D], Wc1[D:2 * D])
    out = _sc_classifier(A, B, C, Wc2[:, 0], row, col)
    return out + bc2[0]


# speedup vs baseline: 1.0660x; 1.0440x over previous
import jax
import jax.numpy as jnp
from jax import lax
from jax.experimental import pallas as pl
from jax.experimental.pallas import tpu as pltpu
from jax.experimental.pallas import tpu_sc as plsc

N = 10000
E = 320000
D = 128
ED = 16

NC = 2
NS = 16
NT = NC * NS
EPT = E // NT
NPAD = 10112
RPS = NPAD // NS

_f32 = jnp.float32
_i32 = jnp.int32


def _p0_body(x_ref, w_ref, asd_ref, h_ref, hsd_ref):
    h = jnp.dot(x_ref[...], w_ref[...], preferred_element_type=_f32)
    h_ref[...] = h
    hsd_ref[...] = jnp.dot(h, asd_ref[...], preferred_element_type=_f32)


def _tc_prep(x, W, a_s, a_d):
    asd = jnp.stack([a_s, a_d], axis=1)
    rb = 128
    nb = NPAD // rb
    return pl.pallas_call(
        _p0_body,
        grid=(nb,),
        in_specs=[
            pl.BlockSpec((rb, D), lambda i: (i, 0)),
            pl.BlockSpec((D, D), lambda i: (0, 0)),
            pl.BlockSpec((D, 2), lambda i: (0, 0)),
        ],
        out_specs=[
            pl.BlockSpec((rb, D), lambda i: (i, 0)),
            pl.BlockSpec((rb, 2), lambda i: (i, 0)),
        ],
        out_shape=[
            jax.ShapeDtypeStruct((NPAD, D), _f32),
            jax.ShapeDtypeStruct((NPAD, 2), _f32),
        ],
    )(x, W, asd)


def _p1_body(ea_ref, we1_ref, ae1_ref, we2_ref, ae2_ref, wc1e_ref, bc1_ref,
             eat_ref, c_ref):
    ve1 = jnp.dot(we1_ref[...], ae1_ref[...], preferred_element_type=_f32)
    ve2 = jnp.dot(we2_ref[...], ae2_ref[...], preferred_element_type=_f32)
    ve = jnp.stack([ve1, ve2], axis=0)
    ea = ea_ref[...]
    eat_ref[...] = lax.dot_general(ve, ea, (((1,), (1,)), ((), ())),
                                   preferred_element_type=_f32)
    c_ref[...] = (jnp.dot(ea, wc1e_ref[...], preferred_element_type=_f32)
                  + bc1_ref[...][None, :])


def _tc_edgeprep(edge_attr, We1, ae1, We2, ae2, Wc1e, bc1):
    nb = 100
    eb = E // nb
    return pl.pallas_call(
        _p1_body,
        grid=(nb,),
        in_specs=[
            pl.BlockSpec((eb, ED), lambda i: (i, 0)),
            pl.BlockSpec((ED, D), lambda i: (0, 0)),
            pl.BlockSpec((D,), lambda i: (0,)),
            pl.BlockSpec((ED, D), lambda i: (0, 0)),
            pl.BlockSpec((D,), lambda i: (0,)),
            pl.BlockSpec((ED, D), lambda i: (0, 0)),
            pl.BlockSpec((D,), lambda i: (0,)),
        ],
        out_specs=[
            pl.BlockSpec((2, eb), lambda i: (0, i)),
            pl.BlockSpec((eb, D), lambda i: (i, 0)),
        ],
        out_shape=[
            jax.ShapeDtypeStruct((2, E), _f32),
            jax.ShapeDtypeStruct((E, D), _f32),
        ],
    )(edge_attr, We1, ae1, We2, ae2, Wc1e, bc1)


def _p2_body(num_ref, den_ref, b_ref, w_ref, asd_ref, h_ref, hsd_ref):
    a = num_ref[0] + num_ref[1]
    den = jnp.sum(den_ref[...], axis=0)[:, None]
    x1 = jnp.maximum(a / (den + 1e-16) + b_ref[...][None, :], 0.0)
    h = jnp.dot(x1, w_ref[...], preferred_element_type=_f32)
    h_ref[...] = h
    hsd_ref[...] = jnp.dot(h, asd_ref[...], preferred_element_type=_f32)


def _tc_combine_prep(num, den, b, W, a_s, a_d):
    asd = jnp.stack([a_s, a_d], axis=1)
    rb = 128
    nb = NPAD // rb
    return pl.pallas_call(
        _p2_body,
        grid=(nb,),
        in_specs=[
            pl.BlockSpec((2, rb, D), lambda i: (0, i, 0)),
            pl.BlockSpec((NT, rb), lambda i: (0, i)),
            pl.BlockSpec((D,), lambda i: (0,)),
            pl.BlockSpec((D, D), lambda i: (0, 0)),
            pl.BlockSpec((D, 2), lambda i: (0, 0)),
        ],
        out_specs=[
            pl.BlockSpec((rb, D), lambda i: (i, 0)),
            pl.BlockSpec((rb, 2), lambda i: (i, 0)),
        ],
        out_shape=[
            jax.ShapeDtypeStruct((NPAD, D), _f32),
            jax.ShapeDtypeStruct((NPAD, 2), _f32),
        ],
    )(num, den, b, W, asd)


def _p3_body(num_ref, den_ref, b_ref, wa_ref, wb_ref, a_ref, bb_ref):
    a = num_ref[0] + num_ref[1]
    den = jnp.sum(den_ref[...], axis=0)[:, None]
    x2 = jnp.maximum(a / (den + 1e-16) + b_ref[...][None, :], 0.0)
    a_ref[...] = jnp.dot(x2, wa_ref[...], preferred_element_type=_f32)
    bb_ref[...] = jnp.dot(x2, wb_ref[...], preferred_element_type=_f32)


def _tc_combine_cls(num, den, b, Wc1a, Wc1b):
    rb = 128
    nb = NPAD // rb
    return pl.pallas_call(
        _p3_body,
        grid=(nb,),
        in_specs=[
            pl.BlockSpec((2, rb, D), lambda i: (0, i, 0)),
            pl.BlockSpec((NT, rb), lambda i: (0, i)),
            pl.BlockSpec((D,), lambda i: (0,)),
            pl.BlockSpec((D, D), lambda i: (0, 0)),
            pl.BlockSpec((D, D), lambda i: (0, 0)),
        ],
        out_specs=[
            pl.BlockSpec((rb, D), lambda i: (i, 0)),
            pl.BlockSpec((rb, D), lambda i: (i, 0)),
        ],
        out_shape=[
            jax.ShapeDtypeStruct((NPAD, D), _f32),
            jax.ShapeDtypeStruct((NPAD, D), _f32),
        ],
    )(num, den, b, Wc1a, Wc1b)


CHW = 128
NCHF = EPT // CHW
TAIL = EPT - NCHF * CHW
NDEN = 10016
CHC = 64
NCHC = EPT // CHC
TAILC = EPT - NCHC * CHC

_mesh = plsc.VectorSubcoreMesh(core_axis_name="c", subcore_axis_name="s",
                               num_cores=NC, num_subcores=NS)
_scp = pltpu.CompilerParams(needs_layout_passes=False)


def _w_body(hs_hbm, hd_hbm, ea_hbm, row_hbm, col_hbm, zerosn_hbm,
            w_out, denp_out,
            hs_v, hd_v, den_v,
            ridx0, ridx1, cval0, cval1, eav0, eav1, wv0, wv1,
            isem0, isem1, osem0, osem1):
    cid = lax.axis_index("c")
    sid = lax.axis_index("s")
    wid = cid * NS + sid
    tbase = wid * EPT
    ridx = (ridx0, ridx1)
    cval = (cval0, cval1)
    eav = (eav0, eav1)
    wv = (wv0, wv1)
    isem = (isem0, isem1)
    osem = (osem0, osem1)

    pltpu.sync_copy(hs_hbm, hs_v)
    pltpu.sync_copy(hd_hbm, hd_v)
    pltpu.sync_copy(zerosn_hbm.at[pl.ds(0, NDEN)], den_v)

    def issue_in(j, p):
        eb = tbase + jnp.minimum(j, NCHF - 1) * CHW
        pltpu.async_copy(row_hbm.at[pl.ds(eb, CHW)], ridx[p], isem[p])
        pltpu.async_copy(col_hbm.at[pl.ds(eb, CHW)], cval[p], isem[p])
        pltpu.async_copy(ea_hbm.at[pl.ds(eb, CHW)], eav[p], isem[p])

    def wait_in(p):
        pltpu.make_async_copy(row_hbm.at[pl.ds(0, CHW)], ridx[p],
                              isem[p]).wait()
        pltpu.make_async_copy(col_hbm.at[pl.ds(0, CHW)], cval[p],
                              isem[p]).wait()
        pltpu.make_async_copy(ea_hbm.at[pl.ds(0, CHW)], eav[p],
                              isem[p]).wait()

    def compute(p, ngroups):
        for k in range(ngroups):
            r16 = ridx[p][pl.ds(k * 16, 16)]
            c16 = cval[p][pl.ds(k * 16, 16)]
            hs = plsc.load_gather(hs_v, [r16])
            hd = plsc.load_gather(hd_v, [c16])
            al = hs + hd + eav[p][pl.ds(k * 16, 16)]
            al = jnp.maximum(al, al * 0.2)
            w16 = jnp.exp(al)
            wv[p][pl.ds(k * 16, 16)] = w16
            plsc.addupdate_scatter(den_v, [c16], w16)

    def issue_out(j, p):
        eb = tbase + j * CHW
        pltpu.async_copy(wv[p], w_out.at[pl.ds(eb, CHW)], osem[p])

    def wait_out(p):
        pltpu.make_async_copy(wv[p], w_out.at[pl.ds(0, CHW)], osem[p]).wait()

    issue_in(0, 0)
    issue_in(1, 1)

    def body(t, carry):
        a = 2 * t
        b = a + 1
        wait_in(0)

        @pl.when(t >= 1)
        def _():
            wait_out(0)

        compute(0, CHW // 16)
        issue_out(a, 0)
        issue_in(a + 2, 0)
        wait_in(1)

        @pl.when(t >= 1)
        def _():
            wait_out(1)

        compute(1, CHW // 16)
        issue_out(b, 1)
        issue_in(b + 2, 1)
        return carry

    lax.fori_loop(0, NCHF // 2, body, 0)
    wait_out(0)
    wait_out(1)
    wait_in(0)
    wait_in(1)

    tb = tbase + NCHF * CHW
    pltpu.sync_copy(row_hbm.at[pl.ds(tb, TAIL)], ridx0.at[pl.ds(0, TAIL)])
    pltpu.sync_copy(col_hbm.at[pl.ds(tb, TAIL)], cval0.at[pl.ds(0, TAIL)])
    pltpu.sync_copy(ea_hbm.at[pl.ds(tb, TAIL)], eav0.at[pl.ds(0, TAIL)])
    compute(0, TAIL // 16)
    pltpu.sync_copy(wv0.at[pl.ds(0, TAIL)], w_out.at[pl.ds(tb, TAIL)])

    pltpu.sync_copy(den_v, denp_out.at[pl.ds(wid * NPAD, NDEN)])


def _sc_weights(hs, hd, ea, row, col, zeros_np):
    k = pl.kernel(
        _w_body,
        out_type=(jax.ShapeDtypeStruct((E,), _f32),
                  jax.ShapeDtypeStruct((NT * NPAD,), _f32)),
        mesh=_mesh,
        scratch_types=[
            pltpu.VMEM((N,), _f32),
            pltpu.VMEM((N,), _f32),
            pltpu.VMEM((NDEN,), _f32),
            pltpu.VMEM((CHW,), _i32), pltpu.VMEM((CHW,), _i32),
            pltpu.VMEM((CHW,), _i32), pltpu.VMEM((CHW,), _i32),
            pltpu.VMEM((CHW,), _f32), pltpu.VMEM((CHW,), _f32),
            pltpu.VMEM((CHW,), _f32), pltpu.VMEM((CHW,), _f32),
            pltpu.SemaphoreType.DMA, pltpu.SemaphoreType.DMA,
            pltpu.SemaphoreType.DMA, pltpu.SemaphoreType.DMA,
        ],
        compiler_params=_scp,
    )
    return k(hs, hd, ea, row, col, zeros_np)


def _m_body(h_hbm, w_hbm, row_hbm, col_hbm, zeros_hbm, acc_out,
            ridx0, ridx1, cval0, cval1, scidx0, scidx1, wval0, wval1,
            hrow0, hrow1, tidx_v, acc_sh,
            isem0, isem1, gsem0, gsem1, ssem0, ssem1):
    cid = lax.axis_index("c")
    sid = lax.axis_index("s")
    wid = cid * NS + sid
    tbase = wid * EPT
    ridx = (ridx0, ridx1)
    cval = (cval0, cval1)
    scidx = (scidx0, scidx1)
    wval = (wval0, wval1)
    hrow = (hrow0, hrow1)
    isem = (isem0, isem1)
    gsem = (gsem0, gsem1)
    ssem = (ssem0, ssem1)

    pltpu.sync_copy(zeros_hbm, acc_sh.at[pl.ds(sid * RPS, RPS)])
    plsc.subcore_barrier()

    def issue_in(j, p):
        eb = tbase + jnp.minimum(j, NCHF - 1) * CHW
        pltpu.async_copy(row_hbm.at[pl.ds(eb, CHW)], ridx[p], isem[p])
        pltpu.async_copy(col_hbm.at[pl.ds(eb, CHW)], cval[p], isem[p])
        pltpu.async_copy(w_hbm.at[pl.ds(eb, CHW)], wval[p], isem[p])

    def wait_in(p):
        pltpu.make_async_copy(row_hbm.at[pl.ds(0, CHW)], ridx[p],
                              isem[p]).wait()
        pltpu.make_async_copy(col_hbm.at[pl.ds(0, CHW)], cval[p],
                              isem[p]).wait()
        pltpu.make_async_copy(w_hbm.at[pl.ds(0, CHW)], wval[p],
                              isem[p]).wait()

    def issue_gather(p):
        pltpu.async_copy(h_hbm.at[ridx[p]], hrow[p], gsem[p])

    def wait_gather(p):
        pltpu.make_async_copy(h_hbm.at[ridx[p]], hrow[p], gsem[p]).wait()

    def process(p, j):
        wregs = [wval[p][pl.ds(k * 16, 16)] for k in range(CHW // 16)]
        for k in range(CHW // 16):
            scidx[p][pl.ds(k * 16, 16)] = cval[p][pl.ds(k * 16, 16)]
        issue_in(j + 2, p)
        for k in range(CHW // 16):
            w16 = wregs[k]
            for i in range(16):
                e = k * 16 + i
                sc = w16[i]
                for c in range(D // 16):
                    v = hrow[p][e, pl.ds(c * 16, 16)]
                    hrow[p][e, pl.ds(c * 16, 16)] = v * sc
        pltpu.async_copy(hrow[p], acc_sh.at[scidx[p]], ssem[p], add=True)

    def wait_scatter(p):
        pltpu.make_async_copy(hrow[p], acc_sh.at[scidx[p]], ssem[p]).wait()

    issue_in(0, 0)
    issue_in(1, 1)
    wait_in(0)
    issue_gather(0)

    def body(t, carry):
        a = 2 * t
        b = a + 1
        wait_in(1)

        @pl.when(t >= 1)
        def _():
            wait_scatter(1)

        issue_gather(1)
        wait_gather(0)
        process(0, a)
        wait_in(0)
        wait_scatter(0)
        issue_gather(0)
        wait_gather(1)
        process(1, b)
        return carry

    lax.fori_loop(0, NCHF // 2, body, 0)
    wait_gather(0)
    wait_scatter(1)
    wait_in(1)

    tb = tbase + NCHF * CHW
    pltpu.sync_copy(row_hbm.at[pl.ds(tb, TAIL)], tidx_v)
    pltpu.sync_copy(col_hbm.at[pl.ds(tb, TAIL)], cval0.at[pl.ds(0, TAIL)])
    pltpu.sync_copy(w_hbm.at[pl.ds(tb, TAIL)], wval0.at[pl.ds(0, TAIL)])
    pltpu.async_copy(h_hbm.at[tidx_v], hrow0.at[pl.ds(0, TAIL)],
                     gsem0).wait()
    w16 = wval0[pl.ds(0, 16)]
    for i in range(TAIL):
        sc = w16[i]
        for c in range(D // 16):
            v = hrow0[i, pl.ds(c * 16, 16)]
            hrow0[i, pl.ds(c * 16, 16)] = v * sc
    tidx_v[pl.ds(0, 16)] = cval0[pl.ds(0, 16)]
    pltpu.sync_copy(hrow0.at[pl.ds(0, TAIL)], acc_sh.at[tidx_v], add=True)

    plsc.subcore_barrier()
    pltpu.sync_copy(acc_sh.at[pl.ds(sid * RPS, RPS)],
                    acc_out.at[cid, pl.ds(sid * RPS, RPS)])


def _sc_aggregate(h, w, row, col, zeros_rps):
    k = pl.kernel(
        _m_body,
        out_type=jax.ShapeDtypeStruct((NC, NPAD, D), _f32),
        mesh=_mesh,
        scratch_types=[
            pltpu.VMEM((CHW,), _i32), pltpu.VMEM((CHW,), _i32),
            pltpu.VMEM((CHW,), _i32), pltpu.VMEM((CHW,), _i32),
            pltpu.VMEM((CHW,), _i32), pltpu.VMEM((CHW,), _i32),
            pltpu.VMEM((CHW,), _f32), pltpu.VMEM((CHW,), _f32),
            pltpu.VMEM((CHW, D), _f32), pltpu.VMEM((CHW, D), _f32),
            pltpu.VMEM((TAIL,), _i32),
            pltpu.VMEM_SHARED((NPAD, D), _f32),
            pltpu.SemaphoreType.DMA, pltpu.SemaphoreType.DMA,
            pltpu.SemaphoreType.DMA, pltpu.SemaphoreType.DMA,
            pltpu.SemaphoreType.DMA, pltpu.SemaphoreType.DMA,
        ],
        compiler_params=_scp,
    )
    return k(h, w, row, col, zeros_rps)


def _cls_body(a_hbm, b_hbm, c_hbm, wc2_hbm, row_hbm, col_hbm, out_hbm,
              wc2_v, tbuf_v, ridx0, ridx1, cval0, cval1,
              arow0, arow1, brow0, brow1, crow0, crow1, outv0, outv1,
              isem0, isem1, gasem0, gasem1, gbsem0, gbsem1,
              csem0, csem1, osem0, osem1):
    cid = lax.axis_index("c")
    sid = lax.axis_index("s")
    wid = cid * NS + sid
    tbase = wid * EPT
    ridx = (ridx0, ridx1)
    cval = (cval0, cval1)
    arow = (arow0, arow1)
    brow = (brow0, brow1)
    crow = (crow0, crow1)
    outv = (outv0, outv1)
    isem = (isem0, isem1)
    gasem = (gasem0, gasem1)
    gbsem = (gbsem0, gbsem1)
    csem = (csem0, csem1)
    osem = (osem0, osem1)

    pltpu.sync_copy(wc2_hbm, wc2_v)
    lane = lax.iota(_i32, 16)
    wc2 = [wc2_v[pl.ds(c * 16, 16)] for c in range(D // 16)]
    jsplat = [jnp.full((16,), j, _i32) for j in range(16)]

    def issue_in(j, p):
        eb = tbase + jnp.minimum(j, NCHC - 1) * CHC
        pltpu.async_copy(row_hbm.at[pl.ds(eb, CHC)], ridx[p], isem[p])
        pltpu.async_copy(col_hbm.at[pl.ds(eb, CHC)], cval[p], isem[p])

    def wait_in(p):
        pltpu.make_async_copy(row_hbm.at[pl.ds(0, CHC)], ridx[p],
                              isem[p]).wait()
        pltpu.make_async_copy(col_hbm.at[pl.ds(0, CHC)], cval[p],
                              isem[p]).wait()

    def issue_gathers(j, p):
        eb = tbase + jnp.minimum(j, NCHC - 1) * CHC
        pltpu.async_copy(a_hbm.at[ridx[p]], arow[p], gasem[p])
        pltpu.async_copy(b_hbm.at[cval[p]], brow[p], gbsem[p])
        pltpu.async_copy(c_hbm.at[pl.ds(eb, CHC)], crow[p], csem[p])

    def wait_gathers(p):
        pltpu.make_async_copy(a_hbm.at[ridx[p]], arow[p], gasem[p]).wait()
        pltpu.make_async_copy(b_hbm.at[cval[p]], brow[p], gbsem[p]).wait()
        pltpu.make_async_copy(c_hbm.at[pl.ds(0, CHC)], crow[p],
                              csem[p]).wait()

    def compute(p, ngroups):
        for k in range(ngroups):
            for i in range(16):
                e = k * 16 + i
                ts = []
                for c in range(D // 16):
                    g = (arow[p][e, pl.ds(c * 16, 16)]
                         + brow[p][e, pl.ds(c * 16, 16)]
                         + crow[p][e, pl.ds(c * 16, 16)])
                    ts.append(jnp.maximum(g, 0.0) * wc2[c])
                acc = (((ts[0] + ts[1]) + (ts[2] + ts[3]))
                       + ((ts[4] + ts[5]) + (ts[6] + ts[7])))
                tbuf_v[i, pl.ds(0, 16)] = acc
            res = plsc.load_gather(tbuf_v, [lane, jsplat[0]])
            for j in range(1, 16):
                res = res + plsc.load_gather(tbuf_v, [lane, jsplat[j]])
            outv[p][pl.ds(k * 16, 16)] = res

    def issue_out(j, p):
        eb = tbase + j * CHC
        pltpu.async_copy(outv[p], out_hbm.at[pl.ds(eb, CHC)], osem[p])

    def wait_out(p):
        pltpu.make_async_copy(outv[p], out_hbm.at[pl.ds(0, CHC)],
                              osem[p]).wait()

    issue_in(0, 0)
    issue_in(1, 1)
    wait_in(0)
    issue_gathers(0, 0)

    def body(t, carry):
        a = 2 * t
        b = a + 1
        wait_in(1)
        issue_gathers(b, 1)
        wait_gathers(0)
        issue_in(a + 2, 0)

        @pl.when(t >= 1)
        def _():
            wait_out(0)

        compute(0, CHC // 16)
        issue_out(a, 0)
        wait_in(0)
        issue_gathers(a + 2, 0)
        wait_gathers(1)
        issue_in(b + 2, 1)

        @pl.when(t >= 1)
        def _():
            wait_out(1)

        compute(1, CHC // 16)
        issue_out(b, 1)
        return carry

    lax.fori_loop(0, NCHC // 2, body, 0)
    wait_gathers(0)
    wait_out(0)
    wait_out(1)
    wait_in(1)

    tb = tbase + NCHC * CHC
    pltpu.sync_copy(row_hbm.at[pl.ds(tb, TAILC)], ridx0.at[pl.ds(0, TAILC)])
    pltpu.sync_copy(col_hbm.at[pl.ds(tb, TAILC)], cval0.at[pl.ds(0, TAILC)])
    cpa = pltpu.async_copy(a_hbm.at[ridx0.at[pl.ds(0, TAILC)]],
                           arow0.at[pl.ds(0, TAILC)], gasem0)
    cpb = pltpu.async_copy(b_hbm.at[cval0.at[pl.ds(0, TAILC)]],
                           brow0.at[pl.ds(0, TAILC)], gbsem0)
    pltpu.sync_copy(c_hbm.at[pl.ds(tb, TAILC)], crow0.at[pl.ds(0, TAILC)])
    cpa.wait()
    cpb.wait()
    compute(0, TAILC // 16)
    pltpu.sync_copy(outv0.at[pl.ds(0, TAILC)], out_hbm.at[pl.ds(tb, TAILC)])


def _sc_classifier(A, B, C, wc2, row, col):
    k = pl.kernel(
        _cls_body,
        out_type=jax.ShapeDtypeStruct((E,), _f32),
        mesh=_mesh,
        scratch_types=[
            pltpu.VMEM((D,), _f32),
            pltpu.VMEM((16, 16), _f32),
            pltpu.VMEM((CHC,), _i32), pltpu.VMEM((CHC,), _i32),
            pltpu.VMEM((CHC,), _i32), pltpu.VMEM((CHC,), _i32),
            pltpu.VMEM((CHC, D), _f32), pltpu.VMEM((CHC, D), _f32),
            pltpu.VMEM((CHC, D), _f32), pltpu.VMEM((CHC, D), _f32),
            pltpu.VMEM((CHC, D), _f32), pltpu.VMEM((CHC, D), _f32),
            pltpu.VMEM((CHC,), _f32), pltpu.VMEM((CHC,), _f32),
            pltpu.SemaphoreType.DMA, pltpu.SemaphoreType.DMA,
            pltpu.SemaphoreType.DMA, pltpu.SemaphoreType.DMA,
            pltpu.SemaphoreType.DMA, pltpu.SemaphoreType.DMA,
            pltpu.SemaphoreType.DMA, pltpu.SemaphoreType.DMA,
            pltpu.SemaphoreType.DMA, pltpu.SemaphoreType.DMA,
        ],
        compiler_params=_scp,
    )
    return k(A, B, C, wc2, row, col)


def kernel(x, edge_index, edge_attr, W1, as1, ad1, We1, ae1, b1,
           W2, as2, ad2, We2, ae2, b2, Wc1, bc1, Wc2, bc2):
    row = edge_index[0].astype(_i32)
    col = edge_index[1].astype(_i32)
    zeros_rps = jnp.zeros((RPS, D), _f32)
    zeros_np = jnp.zeros((NPAD,), _f32)
    x = jnp.pad(x, ((0, NPAD - N), (0, 0)))

    eaT, C = _tc_edgeprep(edge_attr, We1, ae1, We2, ae2, Wc1[2 * D:], bc1)

    h1, hsd1 = _tc_prep(x, W1, as1, ad1)
    w1, denp1 = _sc_weights(hsd1[:N, 0], hsd1[:N, 1], eaT[0], row, col,
                            zeros_np)
    acc1 = _sc_aggregate(h1, w1, row, col, zeros_rps)
    den1 = denp1.reshape(NT, NPAD)

    h2, hsd2 = _tc_combine_prep(acc1, den1, b1, W2, as2, ad2)
    w2, denp2 = _sc_weights(hsd2[:N, 0], hsd2[:N, 1], eaT[1], row, col,
                            zeros_np)
    acc2 = _sc_aggregate(h2, w2, row, col, zeros_rps)
    den2 = denp2.reshape(NT, NPAD)

    A, B = _tc_combine_cls(acc2, den2, b2, Wc1[:D], Wc1[D:2 * D])
    out = _sc_classifier(A, B, C, Wc2[:, 0], row, col)
    return out + bc2[0]
